# baseline (speedup 1.0000x reference)
import functools

import jax
import jax.numpy as jnp
from jax import lax
from jax.experimental import pallas as pl
from jax.experimental.pallas import tpu as pltpu
from jax.experimental.pallas import tpu_sc as plsc

NUM_CORES = 2
NUM_SUBCORES = 16
NW = NUM_CORES * NUM_SUBCORES
LANES = 16
CHUNK = 100
NBUF = 8


def _make_sc_gather_sum(B, KL, D):
    assert B % NW == 0 and D % LANES == 0
    bpw = B // NW
    cpb = KL // CHUNK
    assert cpb * CHUNK == KL and cpb == 2 * NBUF and bpw % 2 == 0
    nj = D // LANES
    mesh = plsc.VectorSubcoreMesh(core_axis_name="c", subcore_axis_name="s")

    @functools.partial(
        pl.kernel,
        mesh=mesh,
        out_type=jax.ShapeDtypeStruct((B, D), jnp.float32),
        scratch_types=[
            pltpu.VMEM((2, cpb, CHUNK), jnp.int32),
            pltpu.VMEM((NBUF, CHUNK, D), jnp.float32),
            pltpu.VMEM((bpw, D), jnp.float32),
        ] + [pltpu.SemaphoreType.DMA] * (NBUF + 2),
    )
    def sc_kernel(idx_hbm, table_hbm, out_hbm, idx_v, bufs, outv, *sems):
        gsems, isems = sems[:NBUF], sems[NBUF:]
        wid = lax.axis_index("s") * NUM_CORES + lax.axis_index("c")
        base_b = wid * bpw

        def idx_start(rb, slot):
            pltpu.async_copy(idx_hbm.at[pl.ds((base_b + rb) * cpb, cpb)],
                             idx_v.at[slot], isems[slot])

        def idx_wait(slot):
            pltpu.make_async_copy(idx_hbm.at[pl.ds(0, cpb)],
                                  idx_v.at[slot], isems[slot]).wait()

        def start(slot, cir, i):
            pltpu.async_copy(table_hbm.at[idx_v.at[slot, cir]],
                             bufs.at[i], gsems[i])

        def wait(i):
            pltpu.make_async_copy(table_hbm.at[idx_v.at[0, 0]],
                                  bufs.at[i], gsems[i]).wait()

        def accum(i, acc):
            def rows4(r4, acc):
                accl = list(acc)
                for rr in range(4):
                    r = r4 * 4 + rr
                    for j in range(nj):
                        accl[j] = accl[j] + bufs[i, r, pl.ds(LANES * j, LANES)]
                return tuple(accl)
            return lax.fori_loop(0, CHUNK // 4, rows4, acc)

        pltpu.sync_copy(idx_hbm.at[pl.ds(base_b * cpb, cpb)], idx_v.at[0])
        idx_start(1, 1)
        for i in range(NBUF):
            start(0, i, i)

        zeros = tuple(jnp.zeros((LANES,), jnp.float32) for _ in range(nj))

        def row_body(rb, sl, acc):
            for i in range(NBUF):
                wait(i)
                acc = accum(i, acc)
                start(sl, NBUF + i, i)

            nxt_ok = rb + 1 < bpw
            for i in range(NBUF):
                wait(i)
                acc = accum(i, acc)
                if i == 0:
                    @pl.when(nxt_ok)
                    def _():
                        idx_wait(1 - sl)

                @pl.when(nxt_ok)
                def _():
                    start(1 - sl, i, i)

            for j in range(nj):
                outv[rb, pl.ds(LANES * j, LANES)] = acc[j]

            @pl.when(rb + 2 < bpw)
            def _():
                idx_start(rb + 2, sl)

            return zeros

        def pair_body(p, acc):
            acc = row_body(2 * p, 0, acc)
            return row_body(2 * p + 1, 1, acc)

        lax.fori_loop(0, bpw // 2, pair_body, zeros)
        pltpu.sync_copy(outv, out_hbm.at[pl.ds(base_b, bpw)])

    return sc_kernel


def _proj_pool_kernel(sums_ref, maskf_ref, w_ref, b_ref, out_ref):
    cnt = jnp.sum(maskf_ref[...], axis=1, keepdims=True)
    denom = jnp.maximum(cnt, 1.0)
    pooled = sums_ref[...] / denom
    proj = lax.dot_general(pooled, w_ref[...], (((1,), (1,)), ((), ())),
                           preferred_element_type=jnp.float32)
    out_ref[...] = proj + b_ref[...] * (cnt / denom)


def kernel(ref_tokens, ref_mask, emb, W, b):
    B, K, L = ref_tokens.shape
    V, D = emb.shape[1], emb.shape[2]
    offs = (jnp.arange(K, dtype=jnp.int32) * V)[None, :, None]
    idx = (ref_tokens.astype(jnp.int32) + offs).reshape(-1, CHUNK)
    table = emb.reshape(K * V, D)

    sums = _make_sc_gather_sum(B, K * L, D)(idx, table)

    maskf = ref_mask.astype(jnp.float32)
    out = pl.pallas_call(
        _proj_pool_kernel,
        out_shape=jax.ShapeDtypeStruct((B, D), jnp.float32),
    )(sums, maskf, W, b.reshape(1, D))
    return out


# speedup vs baseline: 1.0171x; 1.0171x over previous
import functools

import jax
import jax.numpy as jnp
from jax import lax
from jax.experimental import pallas as pl
from jax.experimental.pallas import tpu as pltpu
from jax.experimental.pallas import tpu_sc as plsc

NUM_CORES = 2
NUM_SUBCORES = 16
NW = NUM_CORES * NUM_SUBCORES
LANES = 16
CHUNK = 100
NBUF = 8


def _make_sc_gather_sum(B, KL, D, V, K):
    assert B % NW == 0 and D % LANES == 0
    bpw = B // NW
    cpb = KL // CHUNK
    cpk = cpb // K
    assert cpb * CHUNK == KL and cpb == 2 * NBUF and bpw % 2 == 0
    assert cpk * K == cpb and (KL // K) % CHUNK == 0
    nj = D // LANES
    mesh = plsc.VectorSubcoreMesh(core_axis_name="c", subcore_axis_name="s")

    @functools.partial(
        pl.kernel,
        mesh=mesh,
        out_type=jax.ShapeDtypeStruct((B, D), jnp.float32),
        scratch_types=[
            pltpu.VMEM((2, cpb, CHUNK), jnp.int32),
            pltpu.VMEM((NBUF, CHUNK, D), jnp.float32),
            pltpu.VMEM((bpw, D), jnp.float32),
        ] + [pltpu.SemaphoreType.DMA] * (NBUF + 2),
    )
    def sc_kernel(idx_hbm, table_hbm, out_hbm, idx_v, bufs, outv, *sems):
        gsems, isems = sems[:NBUF], sems[NBUF:]
        wid = lax.axis_index("s") * NUM_CORES + lax.axis_index("c")
        base_b = wid * bpw

        def idx_start(rb, slot):
            pltpu.async_copy(idx_hbm.at[pl.ds((base_b + rb) * cpb, cpb)],
                             idx_v.at[slot], isems[slot])

        def idx_wait(slot):
            pltpu.make_async_copy(idx_hbm.at[pl.ds(0, cpb)],
                                  idx_v.at[slot], isems[slot]).wait()

        def start(slot, cir, i):
            koff = (cir // cpk) * V
            pltpu.async_copy(table_hbm.at[pl.ds(koff, V)].at[idx_v.at[slot, cir]],
                             bufs.at[i], gsems[i])

        def wait(i):
            pltpu.make_async_copy(table_hbm.at[pl.ds(0, V)].at[idx_v.at[0, 0]],
                                  bufs.at[i], gsems[i]).wait()

        def accum(i, acc):
            def rows4(r4, acc):
                accl = list(acc)
                for rr in range(4):
                    r = r4 * 4 + rr
                    for j in range(nj):
                        accl[j] = accl[j] + bufs[i, r, pl.ds(LANES * j, LANES)]
                return tuple(accl)
            return lax.fori_loop(0, CHUNK // 4, rows4, acc)

        pltpu.sync_copy(idx_hbm.at[pl.ds(base_b * cpb, cpb)], idx_v.at[0])
        idx_start(1, 1)
        for i in range(NBUF):
            start(0, i, i)

        zeros = tuple(jnp.zeros((LANES,), jnp.float32) for _ in range(nj))

        def row_body(rb, sl, acc):
            for i in range(NBUF):
                wait(i)
                acc = accum(i, acc)
                start(sl, NBUF + i, i)

            nxt_ok = rb + 1 < bpw
            for i in range(NBUF):
                wait(i)
                acc = accum(i, acc)
                if i == 0:
                    @pl.when(nxt_ok)
                    def _():
                        idx_wait(1 - sl)

                @pl.when(nxt_ok)
                def _():
                    start(1 - sl, i, i)

            for j in range(nj):
                outv[rb, pl.ds(LANES * j, LANES)] = acc[j]

            @pl.when(rb + 2 < bpw)
            def _():
                idx_start(rb + 2, sl)

            return zeros

        def pair_body(p, acc):
            acc = row_body(2 * p, 0, acc)
            return row_body(2 * p + 1, 1, acc)

        lax.fori_loop(0, bpw // 2, pair_body, zeros)
        pltpu.sync_copy(outv, out_hbm.at[pl.ds(base_b, bpw)])

    return sc_kernel


def _proj_pool_kernel(sums_ref, maskf_ref, w_ref, b_ref, out_ref):
    cnt = jnp.sum(maskf_ref[...], axis=1, keepdims=True)
    denom = jnp.maximum(cnt, 1.0)
    pooled = sums_ref[...] / denom
    proj = lax.dot_general(pooled, w_ref[...], (((1,), (1,)), ((), ())),
                           preferred_element_type=jnp.float32)
    out_ref[...] = proj + b_ref[...] * (cnt / denom)


def kernel(ref_tokens, ref_mask, emb, W, b):
    B, K, L = ref_tokens.shape
    V, D = emb.shape[1], emb.shape[2]
    idx = ref_tokens.astype(jnp.int32).reshape(-1, CHUNK)
    table = emb.reshape(K * V, D)

    sums = _make_sc_gather_sum(B, K * L, D, V, K)(idx, table)

    maskf = ref_mask.astype(jnp.float32)
    out = pl.pallas_call(
        _proj_pool_kernel,
        out_shape=jax.ShapeDtypeStruct((B, D), jnp.float32),
    )(sums, maskf, W, b.reshape(1, D))
    return out
